# baseline (device time: 80482 ns/iter reference)
import jax
import jax.numpy as jnp
from jax import lax
from jax.experimental import pallas as pl
from jax.experimental.pallas import tpu as pltpu

N_DEV = 4
B = 2
SQ_LOC = 512
D_MODEL = 768
SKV = 512
HQ = 32
DH = 64
H_LOC = HQ // N_DEV
HD_LOC = H_LOC * DH

_SLOT_OFF = (0, 3, 1, 2)


def kernel(x, Wq, K_ext, V_ext, Wo):
    xb = x.reshape(B * SQ_LOC, D_MODEL).astype(jnp.bfloat16)
    wq_b = (Wq * 0.125).astype(jnp.bfloat16)
    wo_t = Wo.T.astype(jnp.bfloat16)
    k_t = jnp.transpose(K_ext, (0, 2, 1, 3)).astype(jnp.bfloat16)
    v_t = jnp.transpose(V_ext, (0, 2, 1, 3)).astype(jnp.bfloat16)

    def body(x_ref, wq_ref, k_ref, v_ref, wo_ref, out_ref,
             buf, send_sems, recv_sems):
        my = lax.axis_index("i")
        right = lax.rem(my + 1, N_DEV)
        left = lax.rem(my + N_DEV - 1, N_DEV)
        diag = lax.rem(my + 2, N_DEV)

        barrier = pltpu.get_barrier_semaphore()
        for nbr in (left, right, diag):
            pl.semaphore_signal(barrier, inc=1, device_id=(nbr,),
                                device_id_type=pl.DeviceIdType.MESH)
        pl.semaphore_wait(barrier, 3)

        buf[0, 0] = wq_ref[...]
        buf[0, 1] = wo_ref[...]
        sends = []
        for i, (dst_dev, dst_slot) in enumerate(
                ((right, 1), (left, 2), (diag, 3))):
            r = pltpu.make_async_remote_copy(
                src_ref=buf.at[0], dst_ref=buf.at[dst_slot],
                send_sem=send_sems.at[i], recv_sem=recv_sems.at[i],
                device_id=(dst_dev,), device_id_type=pl.DeviceIdType.MESH)
            r.start()
            sends.append(r)

        x2 = x_ref[...]
        row = lax.broadcasted_iota(jnp.int32, (SQ_LOC, SKV), 0)
        col = lax.broadcasted_iota(jnp.int32, (SQ_LOC, SKV), 1)
        bias = jnp.where(((row // 64) % 4) == ((col // 64) % 4),
                         0.0, -1e9).astype(jnp.float32)

        acc = None
        for s in range(N_DEV):
            if s > 0:
                pltpu.make_async_remote_copy(
                    src_ref=buf.at[0], dst_ref=buf.at[s],
                    send_sem=send_sems.at[s - 1],
                    recv_sem=recv_sems.at[s - 1],
                    device_id=(left,),
                    device_id_type=pl.DeviceIdType.MESH).wait_recv()
            o_s = lax.rem(my + _SLOT_OFF[s], N_DEV)
            q2 = lax.dot_general(
                x2, buf[s, 0], (((1,), (0,)), ((), ())),
                preferred_element_type=jnp.float32).astype(jnp.bfloat16)
            ctx_rows = []
            for b in range(B):
                ctx_heads = []
                for hh in range(H_LOC):
                    qbh = q2[b * SQ_LOC:(b + 1) * SQ_LOC,
                             hh * DH:(hh + 1) * DH]
                    kbh = k_ref[b, o_s * H_LOC + hh]
                    sc = lax.dot_general(
                        qbh, kbh, (((1,), (1,)), ((), ())),
                        preferred_element_type=jnp.float32)
                    w = jnp.exp(sc + bias)
                    w = (w / jnp.sum(w, axis=-1, keepdims=True)
                         ).astype(jnp.bfloat16)
                    vbh = v_ref[b, o_s * H_LOC + hh]
                    ctx_heads.append(lax.dot_general(
                        w, vbh, (((1,), (0,)), ((), ())),
                        preferred_element_type=jnp.float32))
                ctx_rows.append(jnp.concatenate(ctx_heads, axis=1))
            ctx = jnp.concatenate(ctx_rows, axis=0).astype(jnp.bfloat16)
            part = lax.dot_general(
                ctx, buf[s, 1], (((1,), (1,)), ((), ())),
                preferred_element_type=jnp.float32)
            acc = part if acc is None else acc + part
        out_ref[...] = acc.reshape(B, SQ_LOC, D_MODEL)

        for r in sends:
            r.wait_send()

    out = pl.pallas_call(
        body,
        out_shape=jax.ShapeDtypeStruct((B, SQ_LOC, D_MODEL), jnp.float32),
        in_specs=[pl.BlockSpec(memory_space=pltpu.VMEM)] * 5,
        out_specs=pl.BlockSpec(memory_space=pltpu.VMEM),
        scratch_shapes=[
            pltpu.VMEM((N_DEV, 2, D_MODEL, HD_LOC), jnp.bfloat16),
            pltpu.SemaphoreType.DMA((3,)),
            pltpu.SemaphoreType.DMA((3,)),
        ],
        compiler_params=pltpu.CompilerParams(
            collective_id=0, vmem_limit_bytes=100 * 1024 * 1024),
    )(xb, wq_b, k_t, v_t, wo_t)
    return out
